# baseline (speedup 1.0000x reference)
import jax
import jax.numpy as jnp
from jax import lax
from jax.experimental import pallas as pl
from jax.experimental.pallas import tpu as pltpu
from jax.experimental.pallas import tpu_sc as plsc

B = 16384
D = 128

_info = plsc.get_sparse_core_info()
NC = _info.num_cores
NS = _info.num_subcores
L = _info.num_lanes
NW = NC * NS
BPW = B // NW
CH = 64
NCHUNK = BPW // CH


NBUF = 2


def _body(ut, itab, us, its, out, uidx, iidx, ub, ib, outv, stage, sem0, sem1):
    wid = lax.axis_index("s") * NC + lax.axis_index("c")
    base = wid * BPW
    pltpu.sync_copy(us.at[pl.ds(base, BPW)], uidx)
    pltpu.sync_copy(its.at[pl.ds(base, BPW)], iidx)
    sems = (sem0, sem1)

    def start(c):
        slot = c % NBUF
        du = pltpu.async_copy(ut.at[uidx.at[pl.ds(c * CH, CH)]], ub.at[slot],
                              sems[slot])
        di = pltpu.async_copy(itab.at[iidx.at[pl.ds(c * CH, CH)]], ib.at[slot],
                              sems[slot])
        return du, di

    lane17 = lax.iota(jnp.int32, L) * 17
    pend = [start(0)]
    for c in range(NCHUNK):
        slot = c % NBUF
        du, di = pend.pop(0)
        du.wait()
        di.wait()
        if c + 1 < NCHUNK:
            pend.append(start(c + 1))
        ubs = ub.at[slot]
        ibs = ib.at[slot]

        @plsc.parallel_loop(0, CH, 1, unroll=1)
        def _row(r, ubs=ubs, ibs=ibs):
            ps = []
            for k in range(D // L):
                uvk = ubs[r, pl.ds(k * L, L)]
                ivk = ibs[r, pl.ds(k * L, L)]
                ps.append(uvk * ivk)
            while len(ps) > 1:
                ps = [a + b for a, b in zip(ps[::2], ps[1::2])]
            idx = lane17 + (r & (L - 1)) + ((r >> 4) * (L * 17))
            plsc.store_scatter(stage, [idx], ps[0])

        for g in range(CH // L):
            tot = [stage[pl.ds(g * (L * 17) + j * 17, L)] for j in range(L)]
            while len(tot) > 1:
                tot = [a + b for a, b in zip(tot[::2], tot[1::2])]
            sig = 1.0 / (1.0 + jnp.exp(-tot[0]))
            outv[pl.ds(c * CH + g * L, L)] = sig
    pltpu.sync_copy(outv, out.at[pl.ds(base, BPW)])


def kernel(users, items, user_table, item_table):
    k = pl.kernel(
        _body,
        out_type=jax.ShapeDtypeStruct((B,), jnp.float32),
        mesh=plsc.VectorSubcoreMesh(core_axis_name="c", subcore_axis_name="s"),
        compiler_params=pltpu.CompilerParams(needs_layout_passes=False),
        scratch_types=[
            pltpu.VMEM((BPW,), jnp.int32),
            pltpu.VMEM((BPW,), jnp.int32),
            pltpu.VMEM((NBUF, CH, D), jnp.float32),
            pltpu.VMEM((NBUF, CH, D), jnp.float32),
            pltpu.VMEM((BPW,), jnp.float32),
            pltpu.VMEM(((CH // L) * L * 17,), jnp.float32),
            pltpu.SemaphoreType.DMA,
            pltpu.SemaphoreType.DMA,
        ],
    )
    return k(user_table, item_table, users, items)


# speedup vs baseline: 1.0897x; 1.0897x over previous
import jax
import jax.numpy as jnp
from jax import lax
from jax.experimental import pallas as pl
from jax.experimental.pallas import tpu as pltpu
from jax.experimental.pallas import tpu_sc as plsc

B = 16384
D = 128

_info = plsc.get_sparse_core_info()
NC = _info.num_cores
NS = _info.num_subcores
L = _info.num_lanes
NW = NC * NS
BPW = B // NW
CH = 128
NCHUNK = BPW // CH


NBUF = 2


def _body(ut, itab, us, its, out, uidx, iidx, ub, ib, outv, stage, sem0, sem1):
    wid = lax.axis_index("s") * NC + lax.axis_index("c")
    base = wid * BPW
    pltpu.sync_copy(us.at[pl.ds(base, BPW)], uidx)
    pltpu.sync_copy(its.at[pl.ds(base, BPW)], iidx)
    sems = (sem0, sem1)

    def start(c):
        slot = c % NBUF
        du = pltpu.async_copy(ut.at[uidx.at[pl.ds(c * CH, CH)]], ub.at[slot],
                              sems[slot])
        di = pltpu.async_copy(itab.at[iidx.at[pl.ds(c * CH, CH)]], ib.at[slot],
                              sems[slot])
        return du, di

    lane17 = lax.iota(jnp.int32, L) * 17
    pend = [start(0)]
    for c in range(NCHUNK):
        slot = c % NBUF
        du, di = pend.pop(0)
        du.wait()
        di.wait()
        if c + 1 < NCHUNK:
            pend.append(start(c + 1))
        ubs = ub.at[slot]
        ibs = ib.at[slot]

        @plsc.parallel_loop(0, CH, 1, unroll=1)
        def _row(r, ubs=ubs, ibs=ibs):
            ps = []
            for k in range(D // L):
                uvk = ubs[r, pl.ds(k * L, L)]
                ivk = ibs[r, pl.ds(k * L, L)]
                ps.append(uvk * ivk)
            while len(ps) > 1:
                ps = [a + b for a, b in zip(ps[::2], ps[1::2])]
            idx = lane17 + (r & (L - 1)) + ((r >> 4) * (L * 17))
            plsc.store_scatter(stage, [idx], ps[0])

        @plsc.parallel_loop(0, CH // L, 1, unroll=1)
        def _comb(g, c=c):
            tot = [stage[pl.ds(g * (L * 17) + j * 17, L)] for j in range(L)]
            while len(tot) > 1:
                tot = [a + b for a, b in zip(tot[::2], tot[1::2])]
            sig = 1.0 / (1.0 + jnp.exp(-tot[0]))
            outv[pl.ds(c * CH + g * L, L)] = sig
    pltpu.sync_copy(outv, out.at[pl.ds(base, BPW)])


def kernel(users, items, user_table, item_table):
    k = pl.kernel(
        _body,
        out_type=jax.ShapeDtypeStruct((B,), jnp.float32),
        mesh=plsc.VectorSubcoreMesh(core_axis_name="c", subcore_axis_name="s"),
        compiler_params=pltpu.CompilerParams(needs_layout_passes=False),
        scratch_types=[
            pltpu.VMEM((BPW,), jnp.int32),
            pltpu.VMEM((BPW,), jnp.int32),
            pltpu.VMEM((NBUF, CH, D), jnp.float32),
            pltpu.VMEM((NBUF, CH, D), jnp.float32),
            pltpu.VMEM((BPW,), jnp.float32),
            pltpu.VMEM(((CH // L) * L * 17,), jnp.float32),
            pltpu.SemaphoreType.DMA,
            pltpu.SemaphoreType.DMA,
        ],
    )
    return k(user_table, item_table, users, items)


# speedup vs baseline: 1.1056x; 1.0146x over previous
import jax
import jax.numpy as jnp
from jax import lax
from jax.experimental import pallas as pl
from jax.experimental.pallas import tpu as pltpu
from jax.experimental.pallas import tpu_sc as plsc

B = 16384
D = 128

_info = plsc.get_sparse_core_info()
NC = _info.num_cores
NS = _info.num_subcores
L = _info.num_lanes
NW = NC * NS
BPW = B // NW
CH = 128
NCHUNK = BPW // CH


NBUF = 2


def _body(ut, itab, us, its, out, uidx, iidx, ub, ib, outv, stage,
          sem0, sem1, isem, osem):
    wid = lax.axis_index("s") * NC + lax.axis_index("c")
    base = wid * BPW
    idx_cp = [
        (pltpu.async_copy(us.at[pl.ds(base + c * CH, CH)],
                          uidx.at[pl.ds(c * CH, CH)], isem),
         pltpu.async_copy(its.at[pl.ds(base + c * CH, CH)],
                          iidx.at[pl.ds(c * CH, CH)], isem))
        for c in range(NCHUNK)
    ]
    sems = (sem0, sem1)

    def start(c):
        slot = c % NBUF
        cu, ci = idx_cp[c]
        cu.wait()
        ci.wait()
        du = pltpu.async_copy(ut.at[uidx.at[pl.ds(c * CH, CH)]], ub.at[slot],
                              sems[slot])
        di = pltpu.async_copy(itab.at[iidx.at[pl.ds(c * CH, CH)]], ib.at[slot],
                              sems[slot])
        return du, di

    lane17 = lax.iota(jnp.int32, L) * 17
    pend = [start(0)]
    out_cp = []
    for c in range(NCHUNK):
        slot = c % NBUF
        du, di = pend.pop(0)
        du.wait()
        di.wait()
        if c + 1 < NCHUNK:
            pend.append(start(c + 1))
        ubs = ub.at[slot]
        ibs = ib.at[slot]

        @plsc.parallel_loop(0, CH, 1, unroll=1)
        def _row(r, ubs=ubs, ibs=ibs):
            ps = []
            for k in range(D // L):
                uvk = ubs[r, pl.ds(k * L, L)]
                ivk = ibs[r, pl.ds(k * L, L)]
                ps.append(uvk * ivk)
            while len(ps) > 1:
                ps = [a + b for a, b in zip(ps[::2], ps[1::2])]
            idx = lane17 + (r & (L - 1)) + ((r >> 4) * (L * 17))
            plsc.store_scatter(stage, [idx], ps[0])

        @plsc.parallel_loop(0, CH // L, 1, unroll=1)
        def _comb(g, c=c):
            tot = [stage[pl.ds(g * (L * 17) + j * 17, L)] for j in range(L)]
            while len(tot) > 1:
                tot = [a + b for a, b in zip(tot[::2], tot[1::2])]
            sig = 1.0 / (1.0 + jnp.exp(-tot[0]))
            outv[pl.ds(c * CH + g * L, L)] = sig

        out_cp.append(pltpu.async_copy(outv.at[pl.ds(c * CH, CH)],
                                       out.at[pl.ds(base + c * CH, CH)], osem))
    for cp in out_cp:
        cp.wait()


def kernel(users, items, user_table, item_table):
    k = pl.kernel(
        _body,
        out_type=jax.ShapeDtypeStruct((B,), jnp.float32),
        mesh=plsc.VectorSubcoreMesh(core_axis_name="c", subcore_axis_name="s"),
        compiler_params=pltpu.CompilerParams(needs_layout_passes=False),
        scratch_types=[
            pltpu.VMEM((BPW,), jnp.int32),
            pltpu.VMEM((BPW,), jnp.int32),
            pltpu.VMEM((NBUF, CH, D), jnp.float32),
            pltpu.VMEM((NBUF, CH, D), jnp.float32),
            pltpu.VMEM((BPW,), jnp.float32),
            pltpu.VMEM(((CH // L) * L * 17,), jnp.float32),
            pltpu.SemaphoreType.DMA,
            pltpu.SemaphoreType.DMA,
            pltpu.SemaphoreType.DMA,
            pltpu.SemaphoreType.DMA,
        ],
    )
    return k(user_table, item_table, users, items)
